# Initial kernel scaffold; baseline (speedup 1.0000x reference)
import jax
import jax.numpy as jnp
from jax.experimental import pallas as pl


def kernel(features, labels):
    raise NotImplementedError("write your pallas kernel here")


# speedup vs baseline: 2.5521x; 2.5521x over previous
import jax
import jax.numpy as jnp
from jax.experimental import pallas as pl
from jax.experimental.pallas import tpu as pltpu

_TEMP_INV_SQRT = 3.1622776601683795
_NEG = -1e30
_BR = 256
_BN = 512


def _norm_kernel(x_ref, o_ref):
    x = x_ref[...]
    n = jnp.sqrt(jnp.sum(x * x, axis=1, keepdims=True))
    o_ref[...] = x * (_TEMP_INV_SQRT / jnp.maximum(n, 1e-12))


def _loss_kernel(fi_ref, f_ref, lr_ref, lc_ref, ls_ref, cnt_ref):
    i = pl.program_id(0)
    br, n = fi_ref.shape[0], f_ref.shape[0]
    fi = fi_ref[...]
    f = f_ref[...]
    sim = jax.lax.dot_general(fi, f, (((1,), (1,)), ((), ())),
                              preferred_element_type=jnp.float32)

    lcol = lc_ref[...]
    lrow = lr_ref[...]
    fg_col = lcol >= 0
    fg_row = lrow >= 0

    col_bias = jnp.where(fg_col, 0.0, _NEG)
    rid = i * br + jax.lax.broadcasted_iota(jnp.int32, (br, n), 0)
    cid = jax.lax.broadcasted_iota(jnp.int32, (br, n), 1)
    simm = jnp.where(rid == cid, _NEG, sim + col_bias)

    m = jnp.clip(jnp.max(simm, axis=1, keepdims=True), -20.0, 20.0)
    e = jnp.exp(simm - m)
    all_sum = jnp.sum(e, axis=1, keepdims=True)

    lc_eff = jnp.where(fg_col, lcol, -3)
    lr_eff = jnp.where(fg_row, lrow, -2)
    e_pos = jnp.where(lr_eff == lc_eff, e, 0.0)
    pos_sum = jnp.sum(e_pos, axis=1, keepdims=True)

    pos_c = jnp.clip(pos_sum, 1e-6, 1e6)
    all_c = jnp.clip(all_sum, 1e-6, 1e6)
    loss = jnp.minimum(-jnp.log(pos_c / all_c), 10.0)

    valid = jnp.where(fg_row & (pos_sum > 0.0), 1.0, 0.0)
    ls_ref[...] = jnp.full(ls_ref.shape, jnp.sum(loss * valid), jnp.float32)
    cnt_ref[...] = jnp.full(cnt_ref.shape, jnp.sum(valid), jnp.float32)


def kernel(features, labels):
    n, d = features.shape
    labels = labels.astype(jnp.int32)

    fn = pl.pallas_call(
        _norm_kernel,
        out_shape=jax.ShapeDtypeStruct((n, d), jnp.float32),
        grid=(n // _BN,),
        in_specs=[pl.BlockSpec((_BN, d), lambda i: (i, 0))],
        out_specs=pl.BlockSpec((_BN, d), lambda i: (i, 0)),
        compiler_params=pltpu.CompilerParams(
            dimension_semantics=("parallel",)),
        name="cpe_normalize",
    )(features)

    nb = n // _BR
    ls, cnt = pl.pallas_call(
        _loss_kernel,
        out_shape=[jax.ShapeDtypeStruct((nb, 1, 128), jnp.float32),
                   jax.ShapeDtypeStruct((nb, 1, 128), jnp.float32)],
        grid=(nb,),
        in_specs=[
            pl.BlockSpec((_BR, d), lambda i: (i, 0)),
            pl.BlockSpec((n, d), lambda i: (0, 0)),
            pl.BlockSpec((_BR, 1), lambda i: (i, 0)),
            pl.BlockSpec((1, n), lambda i: (0, 0)),
        ],
        out_specs=[pl.BlockSpec((1, 1, 128), lambda i: (i, 0, 0)),
                   pl.BlockSpec((1, 1, 128), lambda i: (i, 0, 0))],
        compiler_params=pltpu.CompilerParams(
            dimension_semantics=("parallel",),
            vmem_limit_bytes=56 * 1024 * 1024),
        name="cpe_loss",
    )(fn, fn, labels.reshape(n, 1), labels.reshape(1, n))

    total = jnp.sum(ls[:, 0, 0])
    n_valid = jnp.sum(cnt[:, 0, 0])
    mean = total / jnp.maximum(n_valid, 1.0)
    return jnp.where(n_valid > 0.0, mean, jnp.float32(0.0))
